# Initial kernel scaffold; baseline (speedup 1.0000x reference)
import jax
import jax.numpy as jnp
from jax.experimental import pallas as pl


def kernel(x, edge_index, edge_attr, batch, atom_emb, bond_emb, eps, W1, b1, bn1_g, bn1_b, W2, b2, bn2_g, bn2_b):
    raise NotImplementedError("write your pallas kernel here")


# speedup vs baseline: 4.4231x; 4.4231x over previous
import functools

import jax
import jax.numpy as jnp
from jax import lax
from jax.experimental import pallas as pl
from jax.experimental.pallas import tpu as pltpu
from jax.experimental.pallas import tpu_sc as plsc

NC = 2
NS = 16
NW = NC * NS
LANES = 16

N = 10000
NODES_PER_W = 320
N_PAD = NW * NODES_PER_W
NODE_CHUNK = 64
N_CHUNKS = NODES_PER_W // NODE_CHUNK

E = 320000
EDGE_CHUNK = 128
E_CHUNKS = 79
EDGES_PER_W = E_CHUNKS * EDGE_CHUNK
E_PAD = NW * EDGES_PER_W

D = 128
NB = D // LANES
G = 64

def _worker_id():
    return lax.axis_index("s") * NC + lax.axis_index("c")


def _atom_encode(idx_hbm, tab_hbm, out_hbm, idxv, g9, acc, sem):
    base = _worker_id() * NODES_PER_W

    def chunk(k, _):
        nb = base + k * NODE_CHUNK
        for j in range(9):
            pltpu.sync_copy(idx_hbm.at[pl.ds(j * N_PAD + nb, NODE_CHUNK)],
                            idxv.at[j])
        descs = [
            pltpu.async_copy(tab_hbm.at[idxv.at[j]],
                             g9.at[pl.ds(j * NODE_CHUNK, NODE_CHUNK)], sem)
            for j in range(9)
        ]
        for dsc in descs:
            dsc.wait()

        def row(i, _):
            for j in range(NB):
                s = pl.ds(j * LANES, LANES)
                v = g9[i, s]
                for col in range(1, 9):
                    v = v + g9[col * NODE_CHUNK + i, s]
                acc[i, s] = v
            return 0

        lax.fori_loop(0, NODE_CHUNK, row, 0)
        pltpu.sync_copy(acc, out_hbm.at[pl.ds(nb, NODE_CHUNK)])
        return 0

    lax.fori_loop(0, N_CHUNKS, chunk, 0)


def _msg_aggr(h_hbm, comb_hbm, src_hbm, cmb_hbm, dst_hbm, out_hbm,
              srcv, cmbv, dstv, hbuf, bbuf, aggr_sh, sem):
    cid = lax.axis_index("c")
    sid = lax.axis_index("s")
    wid = sid * NC + cid
    slab = sid * (N_PAD // NS)

    def zrow(i, _):
        for j in range(NB):
            hbuf[i, pl.ds(j * LANES, LANES)] = jnp.zeros((LANES,), jnp.float32)
        return 0

    lax.fori_loop(0, EDGE_CHUNK, zrow, 0)
    for k in range(N_PAD // NS // EDGE_CHUNK):
        pltpu.sync_copy(hbuf, aggr_sh.at[pl.ds(slab + k * EDGE_CHUNK,
                                               EDGE_CHUNK)])
    plsc.subcore_barrier()

    ebase = wid * EDGES_PER_W

    def chunk(k, _):
        eb = ebase + k * EDGE_CHUNK
        pltpu.sync_copy(src_hbm.at[pl.ds(eb, EDGE_CHUNK)], srcv)
        pltpu.sync_copy(cmb_hbm.at[pl.ds(eb, EDGE_CHUNK)], cmbv)
        pltpu.sync_copy(dst_hbm.at[pl.ds(eb, EDGE_CHUNK)], dstv)
        d1 = pltpu.async_copy(h_hbm.at[srcv], hbuf, sem)
        d2 = pltpu.async_copy(comb_hbm.at[cmbv], bbuf, sem)
        d1.wait()
        d2.wait()

        def row(i, _):
            for j in range(NB):
                s = pl.ds(j * LANES, LANES)
                hbuf[i, s] = jnp.maximum(hbuf[i, s] + bbuf[i, s], 0.0)
            return 0

        lax.fori_loop(0, EDGE_CHUNK, row, 0)
        pltpu.sync_copy(hbuf, aggr_sh.at[dstv], add=True)
        return 0

    lax.fori_loop(0, E_CHUNKS, chunk, 0)
    plsc.subcore_barrier()
    pltpu.sync_copy(aggr_sh.at[pl.ds(slab, N_PAD // NS)],
                    out_hbm.at[cid, pl.ds(slab, N_PAD // NS)])


@functools.lru_cache(maxsize=None)
def _sc_kernels():
    mesh = plsc.VectorSubcoreMesh(core_axis_name="c", subcore_axis_name="s",
                                  num_cores=NC, num_subcores=NS)
    atom = pl.kernel(
        _atom_encode,
        out_type=jax.ShapeDtypeStruct((N_PAD, D), jnp.float32),
        mesh=mesh,
        scratch_types=[
            pltpu.VMEM((9, NODE_CHUNK), jnp.int32),
            pltpu.VMEM((9 * NODE_CHUNK, D), jnp.float32),
            pltpu.VMEM((NODE_CHUNK, D), jnp.float32),
            pltpu.SemaphoreType.DMA,
        ],
    )
    msg = pl.kernel(
        _msg_aggr,
        out_type=jax.ShapeDtypeStruct((NC, N_PAD, D), jnp.float32),
        mesh=mesh,
        scratch_types=[
            pltpu.VMEM((EDGE_CHUNK,), jnp.int32),
            pltpu.VMEM((EDGE_CHUNK,), jnp.int32),
            pltpu.VMEM((EDGE_CHUNK,), jnp.int32),
            pltpu.VMEM((EDGE_CHUNK, D), jnp.float32),
            pltpu.VMEM((EDGE_CHUNK, D), jnp.float32),
            pltpu.VMEM_SHARED((N_PAD, D), jnp.float32),
            pltpu.SemaphoreType.DMA,
        ],
    )
    return atom, msg


def _mlp_body(last, h_ref, p_ref, batch_ref, w1_ref, b1_ref, g1_ref, e1_ref,
              w2_ref, b2_ref, g2_ref, e2_ref, eps_ref, hout_ref, fp_ref):
    h = h_ref[0:N, :]
    z = eps_ref[0, 0] * h + p_ref[0, 0:N, :] + p_ref[1, 0:N, :]
    z1 = jnp.dot(z, w1_ref[:, :], preferred_element_type=jnp.float32) + b1_ref[0, :]
    m = jnp.mean(z1, axis=0, keepdims=True)
    v = jnp.mean((z1 - m) ** 2, axis=0, keepdims=True)
    z1 = jnp.maximum((z1 - m) / jnp.sqrt(v + 1e-5) * g1_ref[0, :] + e1_ref[0, :], 0.0)
    z2 = jnp.dot(z1, w2_ref[:, :], preferred_element_type=jnp.float32) + b2_ref[0, :]
    m = jnp.mean(z2, axis=0, keepdims=True)
    v = jnp.mean((z2 - m) ** 2, axis=0, keepdims=True)
    h2 = (z2 - m) / jnp.sqrt(v + 1e-5) * g2_ref[0, :] + e2_ref[0, :]
    hn = h2 if last else jnp.maximum(h2, 0.0)
    hout_ref[0:N, :] = hn
    hout_ref[N:N_PAD, :] = jnp.zeros((N_PAD - N, D), jnp.float32)
    bm = (batch_ref[0:1, :] == lax.broadcasted_iota(jnp.int32, (G, N), 0)
          ).astype(jnp.float32)
    fp_ref[:, :] = jnp.dot(bm, hn, preferred_element_type=jnp.float32,
                           precision=lax.Precision.HIGHEST)


def _mlp_call(last, h_pad, parts, batch2d, w1, b1, g1, e1, w2, b2, g2, e2, eps1):
    return pl.pallas_call(
        functools.partial(_mlp_body, last),
        out_shape=(jax.ShapeDtypeStruct((N_PAD, D), jnp.float32),
                   jax.ShapeDtypeStruct((G, D), jnp.float32)),
        in_specs=[
            pl.BlockSpec((N_PAD, D), lambda: (0, 0)),
            pl.BlockSpec((NC, N_PAD, D), lambda: (0, 0, 0)),
            pl.BlockSpec((1, N), lambda: (0, 0)),
            pl.BlockSpec((D, D), lambda: (0, 0)),
            pl.BlockSpec((1, D), lambda: (0, 0)),
            pl.BlockSpec((1, D), lambda: (0, 0)),
            pl.BlockSpec((1, D), lambda: (0, 0)),
            pl.BlockSpec((D, D), lambda: (0, 0)),
            pl.BlockSpec((1, D), lambda: (0, 0)),
            pl.BlockSpec((1, D), lambda: (0, 0)),
            pl.BlockSpec((1, D), lambda: (0, 0)),
            pl.BlockSpec(memory_space=pltpu.SMEM),
        ],
        out_specs=(pl.BlockSpec((N_PAD, D), lambda: (0, 0)),
                   pl.BlockSpec((G, D), lambda: (0, 0))),
    )(h_pad, parts, batch2d, w1, b1[None], g1[None], e1[None],
      w2, b2[None], g2[None], e2[None], eps1)


def kernel(x, edge_index, edge_attr, batch, atom_emb, bond_emb, eps,
           W1, b1, bn1_g, bn1_b, W2, b2, bn2_g, bn2_b):
    L = W1.shape[0]

    x32 = x.astype(jnp.int32)
    aidx = (x32 + (jnp.arange(9, dtype=jnp.int32) * 128)[None, :]).T
    aidx_pad = jnp.pad(aidx, ((0, 0), (0, N_PAD - N))).reshape(-1)
    atab = atom_emb.reshape(9 * 128, D)

    ea = edge_attr.astype(jnp.int32)
    cmb_idx = ea[:, 0] + 16 * ea[:, 1] + 256 * ea[:, 2]
    comb = (bond_emb[:, 2][:, :, None, None, :]
            + bond_emb[:, 1][:, None, :, None, :]
            + bond_emb[:, 0][:, None, None, :, :]).reshape(L, 4096, D)
    src_p = jnp.pad(edge_index[0].astype(jnp.int32), (0, E_PAD - E))
    cmb_p = jnp.pad(cmb_idx, (0, E_PAD - E))
    dst_p = jnp.pad(edge_index[1].astype(jnp.int32), (0, E_PAD - E),
                    constant_values=N)
    batch2d = batch.astype(jnp.int32)[None, :]

    atom_k, msg_k = _sc_kernels()
    h_pad = atom_k(aidx_pad, atab)

    fps = []
    for l in range(L):
        parts = msg_k(h_pad, comb[l], src_p, cmb_p, dst_p)
        h_pad, fp = _mlp_call(l == L - 1, h_pad, parts, batch2d,
                              W1[l], b1[l], bn1_g[l], bn1_b[l],
                              W2[l], b2[l], bn2_g[l], bn2_b[l],
                              (1.0 + eps[l]).reshape(1, 1))
        fps.append(fp)

    return h_pad[:N], jnp.stack(fps, axis=1)
